# Initial kernel scaffold; baseline (speedup 1.0000x reference)
import jax
import jax.numpy as jnp
from jax.experimental import pallas as pl


def kernel(x, edge_index, W1, b1, Wmu, bmu, Wlv, blv, Wl, bl):
    raise NotImplementedError("write your pallas kernel here")


# speedup vs baseline: 14.6953x; 14.6953x over previous
import functools

import jax
import jax.numpy as jnp
from jax import lax
from jax.experimental import pallas as pl
from jax.experimental.pallas import tpu as pltpu
from jax.experimental.pallas import tpu_sc as plsc

N = 10000
E = 320000
D_IN = 128
D = 64

NC = 2
NS = 16
NW = NC * NS
CHUNK = 128
CHUNKS = 80
E_PAD = NW * CHUNKS * CHUNK
BIN = N
N_ACC = 10112
RPT = N_ACC // NS

BLK = 400


def _sc_degree(dst3, z16, ones16):
    mesh = plsc.VectorSubcoreMesh(core_axis_name="c", subcore_axis_name="s")

    @functools.partial(
        pl.kernel,
        out_type=jax.ShapeDtypeStruct((NC * N_ACC, 16), jnp.float32),
        mesh=mesh,
        compiler_params=pltpu.CompilerParams(use_tc_tiling_on_sc=False),
        scratch_types=[
            pltpu.VMEM((CHUNKS, CHUNK), jnp.int32),
            pltpu.VMEM((CHUNK, 16), jnp.float32),
            pltpu.VMEM_SHARED((N_ACC, 16), jnp.float32),
        ],
    )
    def deg_kernel(dst_hbm, z_hbm, ones_hbm, out_hbm, dstv, onesv, acc):
        c = lax.axis_index("c")
        s = lax.axis_index("s")
        wid = s * NC + c
        row0 = s * RPT
        pltpu.sync_copy(z_hbm.at[pl.ds(row0, RPT)], acc.at[pl.ds(row0, RPT)])
        pltpu.sync_copy(ones_hbm, onesv)
        pltpu.sync_copy(dst_hbm.at[wid], dstv)
        plsc.subcore_barrier()

        def body(j, carry):
            pltpu.sync_copy(onesv, acc.at[dstv.at[j]], add=True)
            return carry

        lax.fori_loop(0, CHUNKS, body, 0)
        plsc.subcore_barrier()
        pltpu.sync_copy(acc.at[pl.ds(row0, RPT)],
                        out_hbm.at[pl.ds(c * N_ACC + row0, RPT)])

    return deg_kernel(dst3, z16, ones16)


def _sc_message(src3, dst3, u, z64):
    mesh = plsc.VectorSubcoreMesh(core_axis_name="c", subcore_axis_name="s")

    @functools.partial(
        pl.kernel,
        out_type=jax.ShapeDtypeStruct((NC * N_ACC, D), jnp.float32),
        mesh=mesh,
        compiler_params=pltpu.CompilerParams(use_tc_tiling_on_sc=False),
        scratch_types=[
            pltpu.VMEM((CHUNKS + 1, CHUNK), jnp.int32),
            pltpu.VMEM((CHUNKS, CHUNK), jnp.int32),
            pltpu.VMEM((CHUNK, D), jnp.float32),
            pltpu.VMEM((CHUNK, D), jnp.float32),
            pltpu.VMEM_SHARED((N_ACC, D), jnp.float32),
            pltpu.SemaphoreType.DMA,
            pltpu.SemaphoreType.DMA,
        ],
    )
    def msg_kernel(src_hbm, dst_hbm, u_hbm, z_hbm, out_hbm,
                   srcv, dstv, buf0, buf1, acc, sem0, sem1):
        c = lax.axis_index("c")
        s = lax.axis_index("s")
        wid = s * NC + c
        row0 = s * RPT
        pltpu.sync_copy(z_hbm.at[pl.ds(row0, RPT)], acc.at[pl.ds(row0, RPT)])
        pltpu.sync_copy(src_hbm.at[wid], srcv)
        pltpu.sync_copy(dst_hbm.at[wid], dstv)
        plsc.subcore_barrier()

        pltpu.async_copy(u_hbm.at[srcv.at[0]], buf0, sem0)

        def group(g, carry):
            j0 = 2 * g
            pltpu.async_copy(u_hbm.at[srcv.at[j0 + 1]], buf1, sem1)
            pltpu.make_async_copy(u_hbm.at[srcv.at[j0]], buf0, sem0).wait()
            pltpu.sync_copy(buf0, acc.at[dstv.at[j0]], add=True)
            pltpu.async_copy(u_hbm.at[srcv.at[j0 + 2]], buf0, sem0)
            pltpu.make_async_copy(u_hbm.at[srcv.at[j0 + 1]], buf1, sem1).wait()
            pltpu.sync_copy(buf1, acc.at[dstv.at[j0 + 1]], add=True)
            return carry

        lax.fori_loop(0, CHUNKS // 2, group, 0)
        pltpu.make_async_copy(u_hbm.at[srcv.at[CHUNKS]], buf0, sem0).wait()

        plsc.subcore_barrier()
        pltpu.sync_copy(acc.at[pl.ds(row0, RPT)],
                        out_hbm.at[pl.ds(c * N_ACC + row0, RPT)])

    return msg_kernel(src3, dst3, u, z64)


def _tc_stage1(dp, x, W1):
    def body(dp_ref, x_ref, w_ref, u_ref):
        deg = dp_ref[0] + dp_ref[1] + 1.0
        dinv = lax.rsqrt(deg[:, 0:1])
        h = jnp.dot(x_ref[...], w_ref[...],
                    preferred_element_type=jnp.float32,
                    precision=lax.Precision.HIGHEST)
        u_ref[...] = h * dinv

    return pl.pallas_call(
        body,
        grid=(N // BLK,),
        in_specs=[
            pl.BlockSpec((2, BLK, 16), lambda i: (0, i, 0)),
            pl.BlockSpec((BLK, D_IN), lambda i: (i, 0)),
            pl.BlockSpec((D_IN, D), lambda i: (0, 0)),
        ],
        out_specs=pl.BlockSpec((BLK, D), lambda i: (i, 0)),
        out_shape=jax.ShapeDtypeStruct((N, D), jnp.float32),
    )(dp, x, W1)


def _tc_stage2(Sa, u1, dp, b1):
    def body(sa_ref, u1_ref, dp_ref, b1_ref, u2_ref):
        deg = dp_ref[0] + dp_ref[1] + 1.0
        dinv = lax.rsqrt(deg[:, 0:1])
        ssum = sa_ref[0] + sa_ref[1] + u1_ref[...]
        h = jnp.maximum(dinv * ssum + b1_ref[...], 0.0)
        u2_ref[...] = dinv * h

    return pl.pallas_call(
        body,
        grid=(N // BLK,),
        in_specs=[
            pl.BlockSpec((2, BLK, D), lambda i: (0, i, 0)),
            pl.BlockSpec((BLK, D), lambda i: (i, 0)),
            pl.BlockSpec((2, BLK, 16), lambda i: (0, i, 0)),
            pl.BlockSpec((1, D), lambda i: (0, 0)),
        ],
        out_specs=pl.BlockSpec((BLK, D), lambda i: (i, 0)),
        out_shape=jax.ShapeDtypeStruct((N, D), jnp.float32),
    )(Sa, u1, dp, b1)


def _tc_stage3(Sb, u2, dp, Wmu, bmu, Wlv, blv, Wl, bl):
    def body(sb_ref, u2_ref, dp_ref, wmu_ref, bmu_ref, wlv_ref, blv_ref,
             wl_ref, bl_ref, recon_ref, mu_ref, lv_ref):
        deg = dp_ref[0] + dp_ref[1] + 1.0
        dinv = lax.rsqrt(deg[:, 0:1])
        g = dinv * (sb_ref[0] + sb_ref[1] + u2_ref[...])
        mm = functools.partial(jnp.dot, preferred_element_type=jnp.float32,
                               precision=lax.Precision.HIGHEST)
        mu = mm(g, wmu_ref[...]) + bmu_ref[...]
        lv = mm(g, wlv_ref[...]) + blv_ref[...]
        r = mm(mu, wl_ref[...]) + bl_ref[...]
        recon_ref[...] = 1.0 / (1.0 + jnp.exp(-r))
        mu_ref[...] = mu
        lv_ref[...] = lv

    return pl.pallas_call(
        body,
        grid=(N // BLK,),
        in_specs=[
            pl.BlockSpec((2, BLK, D), lambda i: (0, i, 0)),
            pl.BlockSpec((BLK, D), lambda i: (i, 0)),
            pl.BlockSpec((2, BLK, 16), lambda i: (0, i, 0)),
            pl.BlockSpec((D, D), lambda i: (0, 0)),
            pl.BlockSpec((1, D), lambda i: (0, 0)),
            pl.BlockSpec((D, D), lambda i: (0, 0)),
            pl.BlockSpec((1, D), lambda i: (0, 0)),
            pl.BlockSpec((D, D_IN), lambda i: (0, 0)),
            pl.BlockSpec((1, D_IN), lambda i: (0, 0)),
        ],
        out_specs=[
            pl.BlockSpec((BLK, D_IN), lambda i: (i, 0)),
            pl.BlockSpec((BLK, D), lambda i: (i, 0)),
            pl.BlockSpec((BLK, D), lambda i: (i, 0)),
        ],
        out_shape=[
            jax.ShapeDtypeStruct((N, D_IN), jnp.float32),
            jax.ShapeDtypeStruct((N, D), jnp.float32),
            jax.ShapeDtypeStruct((N, D), jnp.float32),
        ],
    )(Sb, u2, dp, Wmu, bmu, Wlv, blv, Wl, bl)


def kernel(x, edge_index, W1, b1, Wmu, bmu, Wlv, blv, Wl, bl):
    src = edge_index[0].astype(jnp.int32)
    dst = edge_index[1].astype(jnp.int32)
    pad = E_PAD - E
    src_p = jnp.concatenate([src, jnp.zeros((pad,), jnp.int32)])
    dst_p = jnp.concatenate([dst, jnp.full((pad,), BIN, jnp.int32)])
    src3 = jnp.concatenate(
        [src_p.reshape(NW, CHUNKS, CHUNK),
         jnp.zeros((NW, 1, CHUNK), jnp.int32)], axis=1)
    dst3 = dst_p.reshape(NW, CHUNKS, CHUNK)

    z16 = jnp.zeros((N_ACC, 16), jnp.float32)
    z64 = jnp.zeros((N_ACC, D), jnp.float32)
    ones16 = jnp.ones((CHUNK, 16), jnp.float32)

    dp = _sc_degree(dst3, z16, ones16).reshape(NC, N_ACC, 16)[:, :N, :]
    u1 = _tc_stage1(dp, x, W1)
    Sa = _sc_message(src3, dst3, u1, z64).reshape(NC, N_ACC, D)[:, :N, :]
    u2 = _tc_stage2(Sa, u1, dp, b1.reshape(1, D))
    Sb = _sc_message(src3, dst3, u2, z64).reshape(NC, N_ACC, D)[:, :N, :]
    recon, mu, logvar = _tc_stage3(Sb, u2, dp, Wmu, bmu.reshape(1, D),
                                   Wlv, blv.reshape(1, D),
                                   Wl, bl.reshape(1, D_IN))
    return (recon, mu, logvar)
